# baseline (speedup 1.0000x reference)
import functools

import jax
import jax.numpy as jnp
from jax import lax
from jax.experimental import pallas as pl
from jax.experimental.pallas import tpu as pltpu
from jax.experimental.pallas import tpu_sc as plsc

_B = 4096
_L = 200
_D = 32
_V = 1000000
_NL = 10
_NC = 2
_NS = 16
_NW = _NC * _NS
_BPW = _B // _NW
_HALF = _D // 16


def _make_sc_pool():
    mesh = plsc.VectorSubcoreMesh(core_axis_name="c", subcore_axis_name="s")

    @functools.partial(
        pl.kernel,
        out_type=jax.ShapeDtypeStruct((_B, _D), jnp.float32),
        mesh=mesh,
        scratch_types=[
            pltpu.VMEM((_BPW, _L), jnp.int32),
            pltpu.VMEM((_L, _BPW), jnp.int32),
            pltpu.VMEM((_BPW, _D), jnp.float32),
            pltpu.SemaphoreType.DMA,
        ],
        compiler_params=pltpu.CompilerParams(
            use_tc_tiling_on_sc=False, needs_layout_passes=False
        ),
    )
    def sc_pool(x_hbm, tab_hbm, out_hbm, xrow_v, idx_v, acc, sem):
        wid = lax.axis_index("s") * _NC + lax.axis_index("c")
        tab2 = tab_hbm
        pltpu.sync_copy(x_hbm.at[wid], xrow_v)

        zero = jnp.zeros((16,), jnp.float32)
        for r in range(_BPW):
            for h in range(_HALF):
                acc[r, pl.ds(16 * h, 16)] = zero

        rows = [
            lax.iota(jnp.int32, 16) + jnp.full((16,), 16 * i, jnp.int32)
            for i in range(_BPW // 16)
        ]

        @pl.loop(0, _L)
        def _transpose(l):
            col = jnp.full((16,), l, jnp.int32)
            for i in range(_BPW // 16):
                v = plsc.load_gather(xrow_v, [rows[i], col])
                idx_v[l, pl.ds(16 * i, 16)] = v

        @pl.loop(0, _L)
        def _fire(l):
            pltpu.async_copy(tab2.at[idx_v.at[l]], acc, sem, add=True)

        @pl.loop(0, _L)
        def _drain(l):
            pltpu.make_async_copy(tab2.at[idx_v.at[0]], acc, sem).wait()

        pltpu.sync_copy(acc, out_hbm.at[pl.ds(wid * _BPW, _BPW)])

    return sc_pool


_sc_pool = _make_sc_pool()


def _head_body(doc_ref, w_ref, b_ref, out_ref):
    out_ref[...] = (
        jnp.dot(doc_ref[...], w_ref[...], preferred_element_type=jnp.float32)
        + b_ref[...]
    )


def _head(doc, W, b2):
    return pl.pallas_call(
        _head_body,
        out_shape=jax.ShapeDtypeStruct((_B, _NL), jnp.float32),
    )(doc, W, b2)


def kernel(x, m, table, W, b):
    del m
    x3 = x.astype(jnp.int32).reshape(_NW, _BPW, _L)
    doc = _sc_pool(x3, table)
    return _head(doc, W, b.reshape(1, _NL))


# speedup vs baseline: 1.0033x; 1.0033x over previous
import functools

import jax
import jax.numpy as jnp
from jax import lax
from jax.experimental import pallas as pl
from jax.experimental.pallas import tpu as pltpu
from jax.experimental.pallas import tpu_sc as plsc

_B = 4096
_L = 200
_D = 32
_V = 1000000
_NL = 10
_NC = 2
_NS = 16
_NW = _NC * _NS
_BPW = _B // _NW
_HALF = _D // 16


def _make_sc_pool():
    mesh = plsc.VectorSubcoreMesh(core_axis_name="c", subcore_axis_name="s")

    @functools.partial(
        pl.kernel,
        out_type=jax.ShapeDtypeStruct((_B, _D), jnp.float32),
        mesh=mesh,
        scratch_types=[
            pltpu.VMEM((_BPW, _L), jnp.int32),
            pltpu.VMEM((_L, _BPW), jnp.int32),
            pltpu.VMEM((_BPW, _D), jnp.float32),
            pltpu.SemaphoreType.DMA,
        ],
        compiler_params=pltpu.CompilerParams(
            use_tc_tiling_on_sc=False, needs_layout_passes=False
        ),
    )
    def sc_pool(x_hbm, tab_hbm, out_hbm, xrow_v, idx_v, acc, sem):
        wid = lax.axis_index("s") * _NC + lax.axis_index("c")
        tab2 = tab_hbm
        pltpu.sync_copy(x_hbm.at[pl.ds(wid * _BPW, _BPW)], xrow_v)

        zero = jnp.zeros((16,), jnp.float32)
        for r in range(_BPW):
            for h in range(_HALF):
                acc[r, pl.ds(16 * h, 16)] = zero

        rows = [
            lax.iota(jnp.int32, 16) + jnp.full((16,), 16 * i, jnp.int32)
            for i in range(_BPW // 16)
        ]

        @pl.loop(0, _L)
        def _transpose(l):
            col = jnp.full((16,), l, jnp.int32)
            for i in range(_BPW // 16):
                v = plsc.load_gather(xrow_v, [rows[i], col])
                idx_v[l, pl.ds(16 * i, 16)] = v

        @pl.loop(0, _L)
        def _fire(l):
            pltpu.async_copy(tab2.at[idx_v.at[l]], acc, sem, add=True)

        @pl.loop(0, _L)
        def _drain(l):
            pltpu.make_async_copy(tab2.at[idx_v.at[0]], acc, sem).wait()

        pltpu.sync_copy(acc, out_hbm.at[pl.ds(wid * _BPW, _BPW)])

    return sc_pool


_sc_pool = _make_sc_pool()


def _head_body(doc_ref, w_ref, b_ref, out_ref):
    out_ref[...] = (
        jnp.dot(doc_ref[...], w_ref[...], preferred_element_type=jnp.float32)
        + b_ref[...]
    )


def _head(doc, W, b2):
    return pl.pallas_call(
        _head_body,
        out_shape=jax.ShapeDtypeStruct((_B, _NL), jnp.float32),
    )(doc, W, b2)


def kernel(x, m, table, W, b):
    del m
    doc = _sc_pool(x.astype(jnp.int32), table)
    return _head(doc, W, b.reshape(1, _NL))


# speedup vs baseline: 1.0124x; 1.0091x over previous
import functools

import jax
import jax.numpy as jnp
from jax import lax
from jax.experimental import pallas as pl
from jax.experimental.pallas import tpu as pltpu
from jax.experimental.pallas import tpu_sc as plsc

_B = 4096
_L = 200
_D = 32
_V = 1000000
_NL = 10
_NC = 2
_NS = 16
_NW = _NC * _NS
_BPW = _B // _NW
_HALF = _D // 16


def _make_sc_pool():
    mesh = plsc.VectorSubcoreMesh(core_axis_name="c", subcore_axis_name="s")

    @functools.partial(
        pl.kernel,
        out_type=jax.ShapeDtypeStruct((_B, _D), jnp.float32),
        mesh=mesh,
        scratch_types=[
            pltpu.VMEM((_L, _BPW), jnp.int32),
            pltpu.VMEM((_BPW, _D), jnp.float32),
            pltpu.SemaphoreType.DMA,
        ],
        compiler_params=pltpu.CompilerParams(
            use_tc_tiling_on_sc=False, needs_layout_passes=False
        ),
    )
    def sc_pool(xt_hbm, tab_hbm, out_hbm, idx_v, acc, sem):
        wid = lax.axis_index("s") * _NC + lax.axis_index("c")
        pltpu.sync_copy(xt_hbm.at[wid], idx_v)

        zero = jnp.zeros((16,), jnp.float32)
        for r in range(_BPW):
            for h in range(_HALF):
                acc[r, pl.ds(16 * h, 16)] = zero

        @pl.loop(0, _L)
        def _fire(l):
            pltpu.async_copy(tab_hbm.at[idx_v.at[l]], acc, sem, add=True)

        @pl.loop(0, _L)
        def _drain(l):
            pltpu.make_async_copy(tab_hbm.at[idx_v.at[0]], acc, sem).wait()

        pltpu.sync_copy(acc, out_hbm.at[pl.ds(wid * _BPW, _BPW)])

    return sc_pool


_sc_pool = _make_sc_pool()


def _xt_body(x_ref, o_ref):
    o_ref[0] = jnp.transpose(x_ref[...], (1, 0))


def _xt(x):
    return pl.pallas_call(
        _xt_body,
        grid=(_NW,),
        in_specs=[pl.BlockSpec((_BPW, _L), lambda w: (w, 0))],
        out_specs=pl.BlockSpec((1, _L, _BPW), lambda w: (w, 0, 0)),
        out_shape=jax.ShapeDtypeStruct((_NW, _L, _BPW), jnp.int32),
    )(x)


def _head_body(doc_ref, w_ref, b_ref, out_ref):
    out_ref[...] = (
        jnp.dot(doc_ref[...], w_ref[...], preferred_element_type=jnp.float32)
        + b_ref[...]
    )


def _head(doc, W, b2):
    return pl.pallas_call(
        _head_body,
        out_shape=jax.ShapeDtypeStruct((_B, _NL), jnp.float32),
    )(doc, W, b2)


def kernel(x, m, table, W, b):
    del m
    xt = _xt(x.astype(jnp.int32))
    doc = _sc_pool(xt, table)
    return _head(doc, W, b.reshape(1, _NL))


# speedup vs baseline: 1.0140x; 1.0016x over previous
import functools

import jax
import jax.numpy as jnp
from jax import lax
from jax.experimental import pallas as pl
from jax.experimental.pallas import tpu as pltpu
from jax.experimental.pallas import tpu_sc as plsc

_B = 4096
_L = 200
_D = 32
_V = 1000000
_NL = 10
_NC = 2
_NS = 16
_NW = _NC * _NS
_BPW = _B // _NW
_HALF = _D // 16


def _make_sc_pool():
    mesh = plsc.VectorSubcoreMesh(core_axis_name="c", subcore_axis_name="s")

    @functools.partial(
        pl.kernel,
        out_type=jax.ShapeDtypeStruct((_B, _D), jnp.float32),
        mesh=mesh,
        scratch_types=[
            pltpu.VMEM((_L, _BPW), jnp.int32),
            pltpu.VMEM((_BPW, _D), jnp.float32),
            pltpu.SemaphoreType.DMA,
        ],
        compiler_params=pltpu.CompilerParams(
            use_tc_tiling_on_sc=False, needs_layout_passes=False
        ),
    )
    def sc_pool(xt_hbm, tab_hbm, out_hbm, idx_v, acc, sem):
        wid = lax.axis_index("s") * _NC + lax.axis_index("c")
        pltpu.sync_copy(xt_hbm.at[pl.ds(wid * _L, _L)], idx_v)

        zero = jnp.zeros((16,), jnp.float32)
        for r in range(_BPW):
            for h in range(_HALF):
                acc[r, pl.ds(16 * h, 16)] = zero

        @pl.loop(0, _L)
        def _fire(l):
            pltpu.async_copy(tab_hbm.at[idx_v.at[l]], acc, sem, add=True)

        @pl.loop(0, _L)
        def _drain(l):
            pltpu.make_async_copy(tab_hbm.at[idx_v.at[0]], acc, sem).wait()

        pltpu.sync_copy(acc, out_hbm.at[pl.ds(wid * _BPW, _BPW)])

    return sc_pool


_sc_pool = _make_sc_pool()


def _xt_body(x_ref, o_ref):
    o_ref[...] = jnp.transpose(x_ref[...], (1, 0))


def _xt(x):
    return pl.pallas_call(
        _xt_body,
        grid=(_NW,),
        in_specs=[pl.BlockSpec((_BPW, _L), lambda w: (w, 0))],
        out_specs=pl.BlockSpec((_L, _BPW), lambda w: (w, 0)),
        out_shape=jax.ShapeDtypeStruct((_NW * _L, _BPW), jnp.int32),
    )(x)


def _head_body(doc_ref, w_ref, b_ref, out_ref):
    out_ref[...] = (
        jnp.dot(doc_ref[...], w_ref[...], preferred_element_type=jnp.float32)
        + b_ref[...]
    )


def _head(doc, W, b2):
    return pl.pallas_call(
        _head_body,
        out_shape=jax.ShapeDtypeStruct((_B, _NL), jnp.float32),
    )(doc, W, b2)


def kernel(x, m, table, W, b):
    del m
    xt = _xt(x.astype(jnp.int32))
    doc = _sc_pool(xt, table)
    return _head(doc, W, b.reshape(1, _NL))
